# initial kernel scaffold (unmeasured)
import jax
import jax.numpy as jnp
from jax import lax
from jax.experimental import pallas as pl
from jax.experimental.pallas import tpu as pltpu


def kernel(
    x,
):
    def body(*refs):
        pass

    out_shape = jax.ShapeDtypeStruct(..., jnp.float32)
    return pl.pallas_call(body, out_shape=out_shape)(...)


# baseline (device time: 219597 ns/iter reference)
import jax
import jax.numpy as jnp
from jax import lax
from jax.experimental import pallas as pl
from jax.experimental.pallas import tpu as pltpu

N_PAIRS = 4
CHUNK = 512


def kernel(x):
    m, n = x.shape
    assert m == 2 * N_PAIRS * CHUNK

    def body(x_hbm, out_ref, xin, send, recv, load_sems, send_sems, recv_sems):
        my_x = lax.axis_index("x")
        my_y = lax.axis_index("y")
        x_nbr = (1 - my_x, my_y)
        y_nbr = (my_x, 1 - my_y)

        barrier_sem = pltpu.get_barrier_semaphore()
        for nbr in (x_nbr, y_nbr):
            pl.semaphore_signal(
                barrier_sem, inc=1,
                device_id=nbr, device_id_type=pl.DeviceIdType.MESH,
            )
        pl.semaphore_wait(barrier_sem, 2)

        def exchange(src_ref, dst_ref, sem_idx, nbr):
            rdma = pltpu.make_async_remote_copy(
                src_ref=src_ref,
                dst_ref=dst_ref,
                send_sem=send_sems.at[sem_idx],
                recv_sem=recv_sems.at[sem_idx],
                device_id=nbr,
                device_id_type=pl.DeviceIdType.MESH,
            )
            rdma.start()
            return rdma

        for p in range(N_PAIRS):
            par = p % 2
            rows_e = pl.ds((2 * p) * CHUNK, CHUNK)
            rows_o = pl.ds((2 * p + 1) * CHUNK, CHUNK)

            ld_e = pltpu.make_async_copy(
                x_hbm.at[rows_e, :], xin.at[0], load_sems.at[0])
            ld_o = pltpu.make_async_copy(
                x_hbm.at[rows_o, :], xin.at[1], load_sems.at[1])
            ld_e.start()
            ld_o.start()
            ld_e.wait()
            ld_o.wait()
            send[0] = xin[0].astype(jnp.bfloat16)
            send[1] = xin[1].astype(jnp.bfloat16)

            r1e = exchange(send.at[0], recv.at[par, 0], par * 4 + 0, x_nbr)
            r1o = exchange(send.at[1], recv.at[par, 1], par * 4 + 1, y_nbr)
            r1e.wait()
            r1o.wait()
            send[0] = send[0] + recv[par, 0]
            send[1] = send[1] + recv[par, 1]

            r2e = exchange(send.at[0], recv.at[par, 2], par * 4 + 2, y_nbr)
            r2o = exchange(send.at[1], recv.at[par, 3], par * 4 + 3, x_nbr)
            r2e.wait()
            r2o.wait()
            out_ref[rows_e, :] = send[0] + recv[par, 2]
            out_ref[rows_o, :] = send[1] + recv[par, 3]

    return pl.pallas_call(
        body,
        out_shape=jax.ShapeDtypeStruct((m, n), jnp.bfloat16),
        in_specs=[pl.BlockSpec(memory_space=pl.ANY)],
        out_specs=pl.BlockSpec(memory_space=pltpu.VMEM),
        scratch_shapes=[
            pltpu.VMEM((2, CHUNK, n), jnp.float32),
            pltpu.VMEM((2, CHUNK, n), jnp.bfloat16),
            pltpu.VMEM((2, 4, CHUNK, n), jnp.bfloat16),
            pltpu.SemaphoreType.DMA((2,)),
            pltpu.SemaphoreType.DMA((8,)),
            pltpu.SemaphoreType.DMA((8,)),
        ],
        compiler_params=pltpu.CompilerParams(collective_id=0),
    )(x)


# device time: 215795 ns/iter; 1.0176x vs baseline; 1.0176x over previous
import jax
import jax.numpy as jnp
from jax import lax
from jax.experimental import pallas as pl
from jax.experimental.pallas import tpu as pltpu

N_PAIRS = 4
CHUNK = 512


def kernel(x):
    m, n = x.shape
    assert m == 2 * N_PAIRS * CHUNK

    def body(x_hbm, out_ref, xin, send, recv, load_sems, send_sems, recv_sems):
        my_x = lax.axis_index("x")
        my_y = lax.axis_index("y")
        x_nbr = (1 - my_x, my_y)
        y_nbr = (my_x, 1 - my_y)

        barrier_sem = pltpu.get_barrier_semaphore()
        for nbr in (x_nbr, y_nbr):
            pl.semaphore_signal(
                barrier_sem, inc=1,
                device_id=nbr, device_id_type=pl.DeviceIdType.MESH,
            )
        pl.semaphore_wait(barrier_sem, 2)

        def start_loads(p):
            par = p % 2
            ld_e = pltpu.make_async_copy(
                x_hbm.at[pl.ds((2 * p) * CHUNK, CHUNK), :],
                xin.at[par, 0], load_sems.at[par * 2 + 0])
            ld_o = pltpu.make_async_copy(
                x_hbm.at[pl.ds((2 * p + 1) * CHUNK, CHUNK), :],
                xin.at[par, 1], load_sems.at[par * 2 + 1])
            ld_e.start()
            ld_o.start()
            return (ld_e, ld_o)

        def exchange(src_ref, dst_ref, sem_idx, nbr):
            rdma = pltpu.make_async_remote_copy(
                src_ref=src_ref,
                dst_ref=dst_ref,
                send_sem=send_sems.at[sem_idx],
                recv_sem=recv_sems.at[sem_idx],
                device_id=nbr,
                device_id_type=pl.DeviceIdType.MESH,
            )
            rdma.start()
            return rdma

        def conv_and_step1(p, loads):
            par = p % 2
            loads[0].wait()
            loads[1].wait()
            send[par, 0] = xin[par, 0].astype(jnp.bfloat16)
            send[par, 1] = xin[par, 1].astype(jnp.bfloat16)
            r1e = exchange(send.at[par, 0], recv.at[par, 0], par * 4 + 0, x_nbr)
            r1o = exchange(send.at[par, 1], recv.at[par, 1], par * 4 + 1, y_nbr)
            return (r1e, r1o)

        loads = {0: start_loads(0)}
        if N_PAIRS > 1:
            loads[1] = start_loads(1)
        step1 = {0: conv_and_step1(0, loads[0])}

        for p in range(N_PAIRS):
            par = p % 2
            if p + 1 < N_PAIRS:
                step1[p + 1] = conv_and_step1(p + 1, loads[p + 1])
            if p + 2 < N_PAIRS:
                loads[p + 2] = start_loads(p + 2)

            r1e, r1o = step1[p]
            r1e.wait()
            r1o.wait()
            send[par, 0] = send[par, 0] + recv[par, 0]
            send[par, 1] = send[par, 1] + recv[par, 1]
            r2e = exchange(send.at[par, 0], recv.at[par, 2], par * 4 + 2, y_nbr)
            r2o = exchange(send.at[par, 1], recv.at[par, 3], par * 4 + 3, x_nbr)

            r2e.wait()
            r2o.wait()
            out_ref[pl.ds((2 * p) * CHUNK, CHUNK), :] = (
                send[par, 0] + recv[par, 2])
            out_ref[pl.ds((2 * p + 1) * CHUNK, CHUNK), :] = (
                send[par, 1] + recv[par, 3])

    return pl.pallas_call(
        body,
        out_shape=jax.ShapeDtypeStruct((m, n), jnp.bfloat16),
        in_specs=[pl.BlockSpec(memory_space=pl.ANY)],
        out_specs=pl.BlockSpec(memory_space=pltpu.VMEM),
        scratch_shapes=[
            pltpu.VMEM((2, 2, CHUNK, n), jnp.float32),
            pltpu.VMEM((2, 2, CHUNK, n), jnp.bfloat16),
            pltpu.VMEM((2, 4, CHUNK, n), jnp.bfloat16),
            pltpu.SemaphoreType.DMA((4,)),
            pltpu.SemaphoreType.DMA((8,)),
            pltpu.SemaphoreType.DMA((8,)),
        ],
        compiler_params=pltpu.CompilerParams(
            collective_id=0,
            vmem_limit_bytes=60 * 1024 * 1024,
        ),
    )(x)


# device time: 182251 ns/iter; 1.2049x vs baseline; 1.1841x over previous
import jax
import jax.numpy as jnp
from jax import lax
from jax.experimental import pallas as pl
from jax.experimental.pallas import tpu as pltpu

N_PAIRS = 4
CHUNK = 512
HALF = CHUNK // 2
QUART = CHUNK // 4


def kernel(x):
    m, n = x.shape
    assert m == 2 * N_PAIRS * CHUNK

    def body(x_hbm, out_ref, xin, stage, r1buf, r2buf,
             load_sems, send_sems, recv_sems):
        my_x = lax.axis_index("x")
        my_y = lax.axis_index("y")
        x_nbr = (1 - my_x, my_y)
        y_nbr = (my_x, 1 - my_y)

        barrier_sem = pltpu.get_barrier_semaphore()
        for nbr in (x_nbr, y_nbr):
            pl.semaphore_signal(
                barrier_sem, inc=1,
                device_id=nbr, device_id_type=pl.DeviceIdType.MESH,
            )
        pl.semaphore_wait(barrier_sem, 2)

        def chunk_geom(p, ci):
            c = 2 * p + ci
            r0 = c * CHUNK
            if ci == 0:
                nbr1, nbr2 = x_nbr, y_nbr
                ax1, ax2 = my_x, my_y
            else:
                nbr1, nbr2 = y_nbr, x_nbr
                ax1, ax2 = my_y, my_x
            keep1 = r0 + ax1 * HALF
            send1 = r0 + (1 - ax1) * HALF
            keepq = keep1 + ax2 * QUART
            sendq = keep1 + (1 - ax2) * QUART
            return nbr1, nbr2, ax1, keep1, send1, keepq, sendq

        def exchange(src_ref, dst_ref, sem_idx, nbr):
            rdma = pltpu.make_async_remote_copy(
                src_ref=src_ref,
                dst_ref=dst_ref,
                send_sem=send_sems.at[sem_idx],
                recv_sem=recv_sems.at[sem_idx],
                device_id=nbr,
                device_id_type=pl.DeviceIdType.MESH,
            )
            rdma.start()
            return rdma

        def start_loads(p):
            par = p % 2
            lds = []
            for ci in (0, 1):
                ld = pltpu.make_async_copy(
                    x_hbm.at[pl.ds((2 * p + ci) * CHUNK, CHUNK), :],
                    xin.at[par, ci], load_sems.at[par * 2 + ci])
                ld.start()
                lds.append(ld)
            return lds

        def conv_and_s1(p, loads):
            par = p % 2
            loads[0].wait()
            loads[1].wait()
            rdmas = []
            for ci in (0, 1):
                nbr1, _, ax1, _, _, _, _ = chunk_geom(p, ci)
                stage[par, ci] = xin[
                    par, ci, pl.ds((1 - ax1) * HALF, HALF), :
                ].astype(jnp.bfloat16)
                rdmas.append(exchange(
                    stage.at[par, ci], r1buf.at[par, ci],
                    par * 8 + ci * 4 + 0, nbr1))
            return rdmas

        loads = {0: start_loads(0)}
        if N_PAIRS > 1:
            loads[1] = start_loads(1)
        s1 = {0: conv_and_s1(0, loads[0])}

        for p in range(N_PAIRS):
            par = p % 2
            if p + 1 < N_PAIRS:
                s1[p + 1] = conv_and_s1(p + 1, loads[p + 1])

            geo = [chunk_geom(p, ci) for ci in (0, 1)]

            s1[p][0].wait()
            s1[p][1].wait()
            for ci in (0, 1):
                _, _, ax1, keep1, _, _, _ = geo[ci]
                kept = xin[
                    par, ci, pl.ds(ax1 * HALF, HALF), :
                ].astype(jnp.bfloat16)
                out_ref[pl.ds(keep1, HALF), :] = kept + r1buf[par, ci]

            if p + 2 < N_PAIRS:
                loads[p + 2] = start_loads(p + 2)

            r2 = []
            for ci in (0, 1):
                _, nbr2, _, _, _, _, sendq = geo[ci]
                r2.append(exchange(
                    out_ref.at[pl.ds(sendq, QUART), :], r2buf.at[par, ci],
                    par * 8 + ci * 4 + 1, nbr2))
            r2[0].wait()
            r2[1].wait()
            for ci in (0, 1):
                _, _, _, _, _, keepq, _ = geo[ci]
                out_ref[pl.ds(keepq, QUART), :] = (
                    out_ref[pl.ds(keepq, QUART), :] + r2buf[par, ci])

            r3 = []
            for ci in (0, 1):
                _, nbr2, _, _, _, keepq, _ = geo[ci]
                r3.append(exchange(
                    out_ref.at[pl.ds(keepq, QUART), :],
                    out_ref.at[pl.ds(keepq, QUART), :],
                    par * 8 + ci * 4 + 2, nbr2))
            r3[0].wait()
            r3[1].wait()

            r4 = []
            for ci in (0, 1):
                nbr1, _, _, keep1, _, _, _ = geo[ci]
                r4.append(exchange(
                    out_ref.at[pl.ds(keep1, HALF), :],
                    out_ref.at[pl.ds(keep1, HALF), :],
                    par * 8 + ci * 4 + 3, nbr1))
            r4[0].wait()
            r4[1].wait()

    return pl.pallas_call(
        body,
        out_shape=jax.ShapeDtypeStruct((m, n), jnp.bfloat16),
        in_specs=[pl.BlockSpec(memory_space=pl.ANY)],
        out_specs=pl.BlockSpec(memory_space=pltpu.VMEM),
        scratch_shapes=[
            pltpu.VMEM((2, 2, CHUNK, n), jnp.float32),
            pltpu.VMEM((2, 2, HALF, n), jnp.bfloat16),
            pltpu.VMEM((2, 2, HALF, n), jnp.bfloat16),
            pltpu.VMEM((2, 2, QUART, n), jnp.bfloat16),
            pltpu.SemaphoreType.DMA((4,)),
            pltpu.SemaphoreType.DMA((16,)),
            pltpu.SemaphoreType.DMA((16,)),
        ],
        compiler_params=pltpu.CompilerParams(
            collective_id=0,
            vmem_limit_bytes=60 * 1024 * 1024,
        ),
    )(x)


# device time: 163137 ns/iter; 1.3461x vs baseline; 1.1172x over previous
import jax
import jax.numpy as jnp
from jax import lax
from jax.experimental import pallas as pl
from jax.experimental.pallas import tpu as pltpu

N_PAIRS = 4
CHUNK = 512
HALF = CHUNK // 2
QUART = CHUNK // 4


def kernel(x):
    m, n = x.shape
    assert m == 2 * N_PAIRS * CHUNK

    def body(x_hbm, out_ref, xin, stage, r1buf, r2buf,
             load_sems, send_sems, recv_sems):
        my_x = lax.axis_index("x")
        my_y = lax.axis_index("y")
        x_nbr = (1 - my_x, my_y)
        y_nbr = (my_x, 1 - my_y)

        barrier_sem = pltpu.get_barrier_semaphore()
        for nbr in (x_nbr, y_nbr):
            pl.semaphore_signal(
                barrier_sem, inc=1,
                device_id=nbr, device_id_type=pl.DeviceIdType.MESH,
            )
        pl.semaphore_wait(barrier_sem, 2)

        def chunk_geom(p, ci):
            r0 = (2 * p + ci) * CHUNK
            if ci == 0:
                nbr1, nbr2 = x_nbr, y_nbr
                ax1, ax2 = my_x, my_y
            else:
                nbr1, nbr2 = y_nbr, x_nbr
                ax1, ax2 = my_y, my_x
            keep1 = r0 + ax1 * HALF
            keepq = keep1 + ax2 * QUART
            sendq = keep1 + (1 - ax2) * QUART
            return nbr1, nbr2, ax1, keep1, keepq, sendq

        def exchange(src_ref, dst_ref, sem_idx, nbr):
            rdma = pltpu.make_async_remote_copy(
                src_ref=src_ref,
                dst_ref=dst_ref,
                send_sem=send_sems.at[sem_idx],
                recv_sem=recv_sems.at[sem_idx],
                device_id=nbr,
                device_id_type=pl.DeviceIdType.MESH,
            )
            rdma.start()
            return rdma

        def start_loads(p):
            par = p % 2
            lds = []
            for ci in (0, 1):
                ld = pltpu.make_async_copy(
                    x_hbm.at[pl.ds((2 * p + ci) * CHUNK, CHUNK), :],
                    xin.at[par, ci], load_sems.at[par * 2 + ci])
                ld.start()
                lds.append(ld)
            return lds

        def sem(p, ci, step):
            return (p % 2) * 8 + ci * 4 + step

        def conv_and_s1(p, loads):
            par = p % 2
            loads[0].wait()
            loads[1].wait()
            rdmas = []
            for ci in (0, 1):
                nbr1, _, ax1, _, _, _ = chunk_geom(p, ci)
                stage[p, ci] = xin[
                    par, ci, pl.ds((1 - ax1) * HALF, HALF), :
                ].astype(jnp.bfloat16)
                rdmas.append(exchange(
                    stage.at[p, ci], r1buf.at[p, ci], sem(p, ci, 0), nbr1))
            return rdmas

        def s1_add(p):
            par = p % 2
            for ci in (0, 1):
                _, _, ax1, keep1, _, _ = chunk_geom(p, ci)
                kept = xin[
                    par, ci, pl.ds(ax1 * HALF, HALF), :
                ].astype(jnp.bfloat16)
                out_ref[pl.ds(keep1, HALF), :] = kept + r1buf[p, ci]

        def start_s2(p):
            return [exchange(
                out_ref.at[pl.ds(chunk_geom(p, ci)[5], QUART), :],
                r2buf.at[p, ci], sem(p, ci, 1), chunk_geom(p, ci)[1])
                for ci in (0, 1)]

        def s2_add(p):
            for ci in (0, 1):
                keepq = chunk_geom(p, ci)[4]
                out_ref[pl.ds(keepq, QUART), :] = (
                    out_ref[pl.ds(keepq, QUART), :] + r2buf[p, ci])

        def start_s3(p):
            return [exchange(
                out_ref.at[pl.ds(chunk_geom(p, ci)[4], QUART), :],
                out_ref.at[pl.ds(chunk_geom(p, ci)[4], QUART), :],
                sem(p, ci, 2), chunk_geom(p, ci)[1])
                for ci in (0, 1)]

        def start_s4(p):
            return [exchange(
                out_ref.at[pl.ds(chunk_geom(p, ci)[3], HALF), :],
                out_ref.at[pl.ds(chunk_geom(p, ci)[3], HALF), :],
                sem(p, ci, 3), chunk_geom(p, ci)[0])
                for ci in (0, 1)]

        loads = {0: start_loads(0)}
        if N_PAIRS > 1:
            loads[1] = start_loads(1)
        s1, s2, s3, s4 = {}, {}, {}, {}

        for t in range(N_PAIRS + 4):
            if t < N_PAIRS:
                s1[t] = conv_and_s1(t, loads[t])
            p = t - 1
            if 0 <= p < N_PAIRS:
                s1[p][0].wait()
                s1[p][1].wait()
                s1_add(p)
                if p + 2 < N_PAIRS:
                    loads[p + 2] = start_loads(p + 2)
                s2[p] = start_s2(p)
            p = t - 2
            if 0 <= p < N_PAIRS:
                s2[p][0].wait()
                s2[p][1].wait()
                s2_add(p)
                s3[p] = start_s3(p)
            p = t - 3
            if 0 <= p < N_PAIRS:
                s3[p][0].wait()
                s3[p][1].wait()
                s4[p] = start_s4(p)
            p = t - 4
            if 0 <= p < N_PAIRS:
                s4[p][0].wait()
                s4[p][1].wait()

    return pl.pallas_call(
        body,
        out_shape=jax.ShapeDtypeStruct((m, n), jnp.bfloat16),
        in_specs=[pl.BlockSpec(memory_space=pl.ANY)],
        out_specs=pl.BlockSpec(memory_space=pltpu.VMEM),
        scratch_shapes=[
            pltpu.VMEM((2, 2, CHUNK, n), jnp.float32),
            pltpu.VMEM((N_PAIRS, 2, HALF, n), jnp.bfloat16),
            pltpu.VMEM((N_PAIRS, 2, HALF, n), jnp.bfloat16),
            pltpu.VMEM((N_PAIRS, 2, QUART, n), jnp.bfloat16),
            pltpu.SemaphoreType.DMA((4,)),
            pltpu.SemaphoreType.DMA((16,)),
            pltpu.SemaphoreType.DMA((16,)),
        ],
        compiler_params=pltpu.CompilerParams(
            collective_id=0,
            vmem_limit_bytes=60 * 1024 * 1024,
        ),
    )(x)


# device time: 156201 ns/iter; 1.4059x vs baseline; 1.0444x over previous
import jax
import jax.numpy as jnp
from jax import lax
from jax.experimental import pallas as pl
from jax.experimental.pallas import tpu as pltpu

PAIR_SZ = (320, 320, 320, 256, 256, 256, 192, 128)
N_PAIRS = len(PAIR_SZ)
BASE = tuple(2 * sum(PAIR_SZ[:p]) for p in range(N_PAIRS))
MAXC = max(PAIR_SZ)


def kernel(x):
    m, n = x.shape
    assert m == 2 * sum(PAIR_SZ)

    def body(x_hbm, out_hbm, xin, stage, r1buf, r2buf, outv,
             load_sems, send_sems, recv_sems, store_sems):
        my_x = lax.axis_index("x")
        my_y = lax.axis_index("y")
        x_nbr = (1 - my_x, my_y)
        y_nbr = (my_x, 1 - my_y)

        barrier_sem = pltpu.get_barrier_semaphore()
        for nbr in (x_nbr, y_nbr):
            pl.semaphore_signal(
                barrier_sem, inc=1,
                device_id=nbr, device_id_type=pl.DeviceIdType.MESH,
            )

        def chunk_geom(p, ci):
            sz = PAIR_SZ[p]
            half, quart = sz // 2, sz // 4
            r0 = BASE[p] + ci * sz
            if ci == 0:
                nbr1, nbr2 = x_nbr, y_nbr
                ax1, ax2 = my_x, my_y
            else:
                nbr1, nbr2 = y_nbr, x_nbr
                ax1, ax2 = my_y, my_x
            keep1 = r0 + ax1 * half
            keepq = keep1 + ax2 * quart
            sendq = keep1 + (1 - ax2) * quart
            return nbr1, nbr2, ax1, keep1, keepq, sendq

        def exchange(src_ref, dst_ref, sem_idx, nbr):
            rdma = pltpu.make_async_remote_copy(
                src_ref=src_ref,
                dst_ref=dst_ref,
                send_sem=send_sems.at[sem_idx],
                recv_sem=recv_sems.at[sem_idx],
                device_id=nbr,
                device_id_type=pl.DeviceIdType.MESH,
            )
            rdma.start()
            return rdma

        def start_loads(p):
            par, sz = p % 2, PAIR_SZ[p]
            lds = []
            for ci in (0, 1):
                ld = pltpu.make_async_copy(
                    x_hbm.at[pl.ds(BASE[p] + ci * sz, sz), :],
                    xin.at[par, ci, pl.ds(0, sz), :],
                    load_sems.at[par * 2 + ci])
                ld.start()
                lds.append(ld)
            return lds

        def sem(p, ci, step):
            return (p % 2) * 8 + ci * 4 + step

        def conv(p, loads):
            par, half = p % 2, PAIR_SZ[p] // 2
            loads[0].wait()
            loads[1].wait()
            for ci in (0, 1):
                ax1 = chunk_geom(p, ci)[2]
                stage[p, ci, pl.ds(0, half), :] = xin[
                    par, ci, pl.ds((1 - ax1) * half, half), :
                ].astype(jnp.bfloat16)

        def start_s1(p):
            half = PAIR_SZ[p] // 2
            return [exchange(
                stage.at[p, ci, pl.ds(0, half), :],
                r1buf.at[p, ci, pl.ds(0, half), :],
                sem(p, ci, 0), chunk_geom(p, ci)[0]) for ci in (0, 1)]

        def s1_add(p):
            par, half = p % 2, PAIR_SZ[p] // 2
            for ci in (0, 1):
                _, _, ax1, keep1, _, _ = chunk_geom(p, ci)
                kept = xin[
                    par, ci, pl.ds(ax1 * half, half), :
                ].astype(jnp.bfloat16)
                outv[pl.ds(keep1, half), :] = (
                    kept + r1buf[p, ci, pl.ds(0, half), :])

        def start_s2(p):
            quart = PAIR_SZ[p] // 4
            return [exchange(
                outv.at[pl.ds(chunk_geom(p, ci)[5], quart), :],
                r2buf.at[p, ci, pl.ds(0, quart), :],
                sem(p, ci, 1), chunk_geom(p, ci)[1])
                for ci in (0, 1)]

        def s2_add(p):
            quart = PAIR_SZ[p] // 4
            for ci in (0, 1):
                keepq = chunk_geom(p, ci)[4]
                outv[pl.ds(keepq, quart), :] = (
                    outv[pl.ds(keepq, quart), :]
                    + r2buf[p, ci, pl.ds(0, quart), :])

        def start_s3(p):
            quart = PAIR_SZ[p] // 4
            return [exchange(
                outv.at[pl.ds(chunk_geom(p, ci)[4], quart), :],
                outv.at[pl.ds(chunk_geom(p, ci)[4], quart), :],
                sem(p, ci, 2), chunk_geom(p, ci)[1])
                for ci in (0, 1)]

        def start_s4(p):
            half = PAIR_SZ[p] // 2
            return [exchange(
                outv.at[pl.ds(chunk_geom(p, ci)[3], half), :],
                outv.at[pl.ds(chunk_geom(p, ci)[3], half), :],
                sem(p, ci, 3), chunk_geom(p, ci)[0])
                for ci in (0, 1)]

        loads = {0: start_loads(0)}
        if N_PAIRS > 1:
            loads[1] = start_loads(1)
        s1, s2, s3, s4 = {}, {}, {}, {}
        stores = {}

        conv(0, loads[0])
        pl.semaphore_wait(barrier_sem, 2)

        for t in range(N_PAIRS + 4):
            if t < N_PAIRS:
                if t > 0:
                    conv(t, loads[t])
                s1[t] = start_s1(t)
            p = t - 1
            if 0 <= p < N_PAIRS:
                s1[p][0].wait()
                s1[p][1].wait()
                s1_add(p)
                if p + 2 < N_PAIRS:
                    loads[p + 2] = start_loads(p + 2)
                s2[p] = start_s2(p)
            p = t - 2
            if 0 <= p < N_PAIRS:
                s2[p][0].wait()
                s2[p][1].wait()
                s2_add(p)
                s3[p] = start_s3(p)
            p = t - 3
            if 0 <= p < N_PAIRS:
                s3[p][0].wait()
                s3[p][1].wait()
                s4[p] = start_s4(p)
            p = t - 4
            if 0 <= p < N_PAIRS:
                s4[p][0].wait()
                s4[p][1].wait()
                st = pltpu.make_async_copy(
                    outv.at[pl.ds(BASE[p], 2 * PAIR_SZ[p]), :],
                    out_hbm.at[pl.ds(BASE[p], 2 * PAIR_SZ[p]), :],
                    store_sems.at[p])
                st.start()
                stores[p] = st

        for p in range(N_PAIRS):
            stores[p].wait()

    return pl.pallas_call(
        body,
        out_shape=jax.ShapeDtypeStruct((m, n), jnp.bfloat16),
        in_specs=[pl.BlockSpec(memory_space=pl.ANY)],
        out_specs=pl.BlockSpec(memory_space=pl.ANY),
        scratch_shapes=[
            pltpu.VMEM((2, 2, MAXC, n), jnp.float32),
            pltpu.VMEM((N_PAIRS, 2, MAXC // 2, n), jnp.bfloat16),
            pltpu.VMEM((N_PAIRS, 2, MAXC // 2, n), jnp.bfloat16),
            pltpu.VMEM((N_PAIRS, 2, MAXC // 4, n), jnp.bfloat16),
            pltpu.VMEM((m, n), jnp.bfloat16),
            pltpu.SemaphoreType.DMA((4,)),
            pltpu.SemaphoreType.DMA((16,)),
            pltpu.SemaphoreType.DMA((16,)),
            pltpu.SemaphoreType.DMA((N_PAIRS,)),
        ],
        compiler_params=pltpu.CompilerParams(
            collective_id=0,
            vmem_limit_bytes=60 * 1024 * 1024,
        ),
    )(x)
